# Initial kernel scaffold; baseline (speedup 1.0000x reference)
import jax
import jax.numpy as jnp
from jax.experimental import pallas as pl


def kernel(x, edge_index, We, be, convW, convb, gamma, beta, W1, b1, W2, b2, W3, b3):
    raise NotImplementedError("write your pallas kernel here")


# speedup vs baseline: 6.7604x; 6.7604x over previous
import functools

import jax
import jax.numpy as jnp
from jax import lax
from jax.experimental import pallas as pl
from jax.experimental.pallas import tpu as pltpu
from jax.experimental.pallas import tpu_sc as plsc

_N = 10000
_E = 160000
_D = 256
_NL = 4
_NS = 16
_NC = 2
_CB = 128
_CH = 80
_IG = 40
_EPAD = _NS * _CH * _CB
_ACCR = 10240
_NPG = _ACCR
_ZR = _ACCR // _NS

_mesh = plsc.VectorSubcoreMesh(core_axis_name="c", subcore_axis_name="s")
_f32 = jnp.float32


@functools.partial(
    pl.kernel,
    out_type=jax.ShapeDtypeStruct((_NC, _ACCR, 128), _f32),
    mesh=_mesh,
    scratch_types=[
        pltpu.VMEM((_CH, _CB), jnp.int32),
        pltpu.VMEM((_CB, 128), _f32),
        pltpu.MemorySpace.VMEM_SHARED((_ACCR, 128), _f32),
    ],
)
def _deg_pass(dst_hbm, ones_hbm, zer_hbm, out_hbm, dst_v, ones_v, acc):
    c = lax.axis_index("c")
    s = lax.axis_index("s")
    pltpu.sync_copy(dst_hbm.at[s], dst_v)
    pltpu.sync_copy(ones_hbm, ones_v)
    pltpu.sync_copy(zer_hbm, acc.at[pl.ds(s * _ZR, _ZR)])
    plsc.subcore_barrier()

    def body(j, carry):
        pltpu.sync_copy(ones_v, acc.at[dst_v.at[j]], add=True)
        return carry

    lax.fori_loop(0, _CH, body, 0)
    plsc.subcore_barrier()
    pltpu.sync_copy(acc.at[pl.ds(s * _ZR, _ZR)],
                    out_hbm.at[c].at[pl.ds(s * _ZR, _ZR)])


@functools.partial(
    pl.kernel,
    out_type=jax.ShapeDtypeStruct((_NC, _ACCR, 128), _f32),
    mesh=_mesh,
    scratch_types=[
        pltpu.VMEM((_IG, _CB), jnp.int32),
        pltpu.VMEM((_IG, _CB), jnp.int32),
        pltpu.VMEM((_CB, 128), _f32),
        pltpu.VMEM((_CB, 128), _f32),
        pltpu.MemorySpace.VMEM_SHARED((_ACCR, 128), _f32),
        pltpu.SemaphoreType.DMA,
        pltpu.SemaphoreType.DMA,
    ],
)
def _edge_pass(g_hbm, src_hbm, dst_hbm, out_hbm,
               src_v, dst_v, buf0, buf1, acc, sem0, sem1):
    c = lax.axis_index("c")
    s = lax.axis_index("s")
    pltpu.sync_copy(g_hbm.at[c].at[pl.ds(s * _ZR, _ZR)],
                    acc.at[pl.ds(s * _ZR, _ZR)])
    plsc.subcore_barrier()

    gh = g_hbm.at[c]
    for grp in range(_CH // _IG):
        pltpu.sync_copy(src_hbm.at[s].at[pl.ds(grp * _IG, _IG)], src_v)
        pltpu.sync_copy(dst_hbm.at[s].at[pl.ds(grp * _IG, _IG)], dst_v)
        pltpu.async_copy(gh.at[src_v.at[0]], buf0, sem0)

        def body(it, carry):
            j = it * 2
            pltpu.async_copy(gh.at[src_v.at[j + 1]], buf1, sem1)
            pltpu.make_async_copy(gh.at[src_v.at[j]], buf0, sem0).wait()
            pltpu.sync_copy(buf0, acc.at[dst_v.at[j]], add=True)

            @pl.when(it < _IG // 2 - 1)
            def _():
                pltpu.async_copy(gh.at[src_v.at[j + 2]], buf0, sem0)

            pltpu.make_async_copy(gh.at[src_v.at[j + 1]], buf1, sem1).wait()
            pltpu.sync_copy(buf1, acc.at[dst_v.at[j + 1]], add=True)
            return carry

        lax.fori_loop(0, _IG // 2, body, 0)
    plsc.subcore_barrier()
    pltpu.sync_copy(acc.at[pl.ds(s * _ZR, _ZR)],
                    out_hbm.at[c].at[pl.ds(s * _ZR, _ZR)])


def _dinv_from(deg_ref):
    deg = deg_ref[0:_N, 0:1] + 1.0
    return lax.rsqrt(jnp.maximum(deg, 1.0))


def _store_g(g_ref, g):
    g_ref[0, 0:_N, :] = g[:, 0:128]
    g_ref[1, 0:_N, :] = g[:, 128:256]
    z = jnp.zeros((_NPG - _N, 128), _f32)
    g_ref[0, _N:_NPG, :] = z
    g_ref[1, _N:_NPG, :] = z


def _bn_core(h_ref, s_ref, dinv, b_ref, gm_ref, bt_ref):
    halves = []
    for c in range(2):
        lo, hi = c * 128, (c + 1) * 128
        agg = dinv * s_ref[c, 0:_N, :] + b_ref[:, lo:hi]
        mu = jnp.mean(agg, axis=0, keepdims=True)
        xc = agg - mu
        var = jnp.mean(xc * xc, axis=0, keepdims=True)
        hbn = xc * lax.rsqrt(var + 1e-5) * gm_ref[:, lo:hi] + bt_ref[:, lo:hi]
        halves.append(h_ref[:, lo:hi] + jnp.maximum(hbn, 0.0))
    return jnp.concatenate(halves, axis=1)


def _emb_body(x_ref, we_ref, be_ref, h_ref):
    h = jnp.dot(x_ref[...], we_ref[...], preferred_element_type=_f32)
    h_ref[...] = h + be_ref[...]


def _mm_body(h_ref, w_ref, deg_ref, g_ref):
    dinv = _dinv_from(deg_ref)
    ht = jnp.dot(h_ref[...], w_ref[...], preferred_element_type=_f32)
    _store_g(g_ref, dinv * ht)


def _bn_body(h_ref, s_ref, deg_ref, b_ref, gm_ref, bt_ref, hn_ref):
    dinv = _dinv_from(deg_ref)
    hn_ref[...] = _bn_core(h_ref, s_ref, dinv, b_ref, gm_ref, bt_ref)


def _fin_body(h_ref, s_ref, deg_ref, b_ref, gm_ref, bt_ref,
              w1_ref, b1_ref, w2_ref, b2_ref, w3_ref, b3_ref, out_ref):
    dinv = _dinv_from(deg_ref)
    hn = _bn_core(h_ref, s_ref, dinv, b_ref, gm_ref, bt_ref)
    r = jnp.dot(hn, w1_ref[...], preferred_element_type=_f32) + b1_ref[...]
    r = jnp.maximum(r, 0.0)
    r = jnp.dot(r, w2_ref[...], preferred_element_type=_f32) + b2_ref[...]
    r = jnp.maximum(r, 0.0)
    out_ref[...] = jnp.dot(r, w3_ref[...], preferred_element_type=_f32) + b3_ref[...]


_sds = jax.ShapeDtypeStruct

_emb_call = pl.pallas_call(
    _emb_body,
    out_shape=_sds((_N, _D), _f32),
)

_mm_call = pl.pallas_call(
    _mm_body,
    out_shape=_sds((_NC, _NPG, 128), _f32),
)

_bn_call = pl.pallas_call(
    _bn_body,
    out_shape=_sds((_N, _D), _f32),
)

_fin_call = pl.pallas_call(
    _fin_body,
    out_shape=_sds((_N, 6), _f32),
)


def kernel(x, edge_index, We, be, convW, convb, gamma, beta,
           W1, b1, W2, b2, W3, b3):
    src = edge_index[0].astype(jnp.int32)
    dst = edge_index[1].astype(jnp.int32)
    pad = jnp.full((_EPAD - _E,), _N, jnp.int32)
    srcp = jnp.concatenate([src, pad]).reshape(_NS, _CH, _CB)
    dstp = jnp.concatenate([dst, pad]).reshape(_NS, _CH, _CB)
    ones_b = jnp.ones((_CB, 128), _f32)
    zer_b = jnp.zeros((_ZR, 128), _f32)

    deg2d = _deg_pass(dstp, ones_b, zer_b)[0]

    be2 = be.reshape(1, _D)
    h = _emb_call(x, We, be2)
    for i in range(_NL - 1):
        g = _mm_call(h, convW[i], deg2d)
        s_agg = _edge_pass(g, srcp, dstp)
        h = _bn_call(h, s_agg, deg2d, convb[i].reshape(1, _D),
                     gamma[i].reshape(1, _D), beta[i].reshape(1, _D))
    i = _NL - 1
    g = _mm_call(h, convW[i], deg2d)
    s_agg = _edge_pass(g, srcp, dstp)
    return _fin_call(h, s_agg, deg2d, convb[i].reshape(1, _D),
                     gamma[i].reshape(1, _D), beta[i].reshape(1, _D),
                     W1, b1.reshape(1, -1), W2, b2.reshape(1, -1),
                     W3, b3.reshape(1, -1))
